# baseline (speedup 1.0000x reference)
import functools

import jax
import jax.numpy as jnp
from jax import lax
from jax.experimental import pallas as pl
from jax.experimental.pallas import tpu as pltpu
from jax.experimental.pallas import tpu_sc as plsc

EMBED = 64
PERIOD = 30
LANES = 16
ROWS = 16384
COLS = 50
N_SLICES = 4
CHUNK_ROWS = 8
CHUNK_POS = CHUNK_ROWS * COLS
CHUNK_PAIRS = CHUNK_POS // 2
SLAB = 3200
GATHER_SPLIT = (128, 72)


def _sc_lookup(pos_flat, ptable, num_workers, n_rows):
    W_ROWS = n_rows // num_workers
    N_CHUNKS = W_ROWS // CHUNK_ROWS
    W_POS = W_ROWS * COLS
    W_PAIRS = W_POS // 2
    N_SLABS = W_POS // SLAB

    mesh = plsc.VectorSubcoreMesh(core_axis_name="c", subcore_axis_name="s")

    @functools.partial(
        pl.kernel,
        out_type=jax.ShapeDtypeStruct((n_rows, COLS, EMBED), jnp.float32),
        mesh=mesh,
        scratch_types=[
            pltpu.VMEM((SLAB,), jnp.int32),
            pltpu.VMEM((W_PAIRS,), jnp.int32),
            pltpu.VMEM((CHUNK_PAIRS, 2 * EMBED), jnp.float32),
            pltpu.VMEM((CHUNK_POS, EMBED), jnp.float32),
            pltpu.SemaphoreType.DMA,
            pltpu.SemaphoreType.DMA,
        ],
    )
    def body(pos_hbm, ptable_hbm, out_hbm, pos_v, pidx_v, g_v, rows3, sem_g,
             sem_o):
        num_cores = lax.axis_size("c")
        wid = lax.axis_index("s") * num_cores + lax.axis_index("c")
        pos_base = wid * W_POS
        row_base = wid * W_ROWS
        lane = lax.iota(jnp.int32, LANES)
        xor1 = lane ^ 1
        evens = (lane * 2) & (LANES - 1)
        lo_half = lane < (LANES // 2)

        def vperm(x, idx):
            return lax.gather(
                x,
                idx[:, None],
                dimension_numbers=lax.GatherDimensionNumbers(
                    offset_dims=(), collapsed_slice_dims=(0,),
                    start_index_map=(0,),
                ),
                slice_sizes=(1,),
                mode=lax.GatherScatterMode.PROMISE_IN_BOUNDS,
            )

        def pair_codes(v):
            r = lax.rem(lax.rem(v, PERIOD) + PERIOD, PERIOD)
            return r * PERIOD + vperm(r, xor1)

        def slab_pass(s, carry):
            off = pl.multiple_of(pos_base + s * SLAB, SLAB)
            pltpu.sync_copy(pos_hbm.at[pl.ds(off, SLAB)], pos_v)

            def group(g, carry2):
                ta = pair_codes(pos_v[pl.ds(g * 2 * LANES, LANES)])
                tb = pair_codes(pos_v[pl.ds(g * 2 * LANES + LANES, LANES)])
                ga = vperm(ta, evens)
                gb = vperm(tb, evens)
                pidx_v[pl.ds(s * (SLAB // 2) + g * LANES, LANES)] = jnp.where(
                    lo_half, ga, gb
                )
                return carry2

            lax.fori_loop(0, SLAB // (2 * LANES), group, 0)
            return carry

        lax.fori_loop(0, N_SLABS, slab_pass, 0)

        def fire_gathers(t):
            o = 0
            for sz in GATHER_SPLIT:
                pltpu.async_copy(
                    ptable_hbm.at[pidx_v.at[pl.ds(t * CHUNK_PAIRS + o, sz)]],
                    g_v.at[pl.ds(o, sz)],
                    sem_g,
                )
                o += sz

        def drain_gathers():
            o = 0
            for sz in GATHER_SPLIT:
                pltpu.make_async_copy(
                    ptable_hbm.at[pidx_v.at[pl.ds(o, sz)]],
                    g_v.at[pl.ds(o, sz)],
                    sem_g,
                ).wait()
                o += sz

        def relayout():
            def per_block(b, carry2):
                p0 = b * 8
                q0 = b * 16
                for dp in range(8):
                    for half in range(2):
                        for l in range(EMBED // LANES):
                            rows3[q0 + 2 * dp + half,
                                  pl.ds(l * LANES, LANES)] = (
                                g_v[p0 + dp,
                                    pl.ds(half * EMBED + l * LANES, LANES)]
                            )
                return carry2

            lax.fori_loop(0, CHUNK_PAIRS // 8, per_block, 0)

        def drain_scatter():
            pltpu.make_async_copy(
                rows3.reshape(CHUNK_ROWS, COLS, EMBED),
                out_hbm.at[pl.ds(0, CHUNK_ROWS)],
                sem_o,
            ).wait()

        fire_gathers(0)

        def step(t, carry):
            drain_gathers()

            @pl.when(t > 0)
            def _():
                drain_scatter()

            relayout()

            @pl.when(t < N_CHUNKS - 1)
            def _():
                fire_gathers(t + 1)

            r0 = pl.multiple_of(row_base + t * CHUNK_ROWS, CHUNK_ROWS)
            pltpu.async_copy(
                rows3.reshape(CHUNK_ROWS, COLS, EMBED),
                out_hbm.at[pl.ds(r0, CHUNK_ROWS)],
                sem_o,
            )
            return carry

        lax.fori_loop(0, N_CHUNKS, step, 0)
        drain_scatter()

    return body(pos_flat, ptable)


def kernel(position, embedding):
    info = plsc.get_sparse_core_info()
    num_workers = info.num_cores * info.num_subcores
    ptable = jnp.concatenate(
        [
            jnp.broadcast_to(embedding[:, None, :], (PERIOD, PERIOD, EMBED)),
            jnp.broadcast_to(embedding[None, :, :], (PERIOD, PERIOD, EMBED)),
        ],
        axis=-1,
    ).reshape(PERIOD * PERIOD, 2 * EMBED)
    pos_flat = position.reshape(-1)
    rows_per_slice = position.shape[0] // N_SLICES
    pos_per_slice = pos_flat.shape[0] // N_SLICES
    parts = [
        _sc_lookup(
            pos_flat[k * pos_per_slice:(k + 1) * pos_per_slice],
            ptable,
            num_workers,
            rows_per_slice,
        )
        for k in range(N_SLICES)
    ]
    return jnp.concatenate(parts, axis=0)


# speedup vs baseline: 1.0978x; 1.0978x over previous
import functools

import jax
import jax.numpy as jnp
from jax import lax
from jax.experimental import pallas as pl
from jax.experimental.pallas import tpu as pltpu
from jax.experimental.pallas import tpu_sc as plsc

EMBED = 64
PERIOD = 30
LANES = 16
CHUNK = 512
PAIRS = CHUNK // 2
IDX_ROWS = PAIRS // 128


def _sc_lookup(pos_flat, ptable, num_workers):
    n = pos_flat.shape[0]
    b_per_w = n // num_workers
    n_chunks = b_per_w // CHUNK

    mesh = plsc.VectorSubcoreMesh(core_axis_name="c", subcore_axis_name="s")

    @functools.partial(
        pl.kernel,
        out_type=jax.ShapeDtypeStruct((n // 2, 2 * EMBED), jnp.float32),
        mesh=mesh,
        scratch_types=[
            pltpu.VMEM((2, CHUNK), jnp.int32),
            pltpu.VMEM((2, IDX_ROWS, 128), jnp.int32),
            pltpu.VMEM((2, PAIRS, 2 * EMBED), jnp.float32),
            pltpu.SemaphoreType.DMA,
            pltpu.SemaphoreType.DMA,
        ],
    )
    def body(pos_hbm, ptable_hbm, out_hbm, pos_v, pidx_v, rows_v, sem_g, sem_o):
        num_cores = lax.axis_size("c")
        wid = lax.axis_index("s") * num_cores + lax.axis_index("c")
        base = wid * b_per_w
        base2 = base // 2
        lane = lax.iota(jnp.int32, LANES)
        xor1 = lane ^ 1
        evens = (lane * 2) & (LANES - 1)
        lo_half = lane < (LANES // 2)

        def vperm(x, idx):
            return lax.gather(
                x,
                idx[:, None],
                dimension_numbers=lax.GatherDimensionNumbers(
                    offset_dims=(), collapsed_slice_dims=(0,),
                    start_index_map=(0,),
                ),
                slice_sizes=(1,),
                mode=lax.GatherScatterMode.PROMISE_IN_BOUNDS,
            )

        def pair_codes(v):
            r = lax.rem(lax.rem(v, PERIOD) + PERIOD, PERIOD)
            return r * PERIOD + vperm(r, xor1)

        def load_and_index(u, buf):
            off = pl.multiple_of(base + u * CHUNK, CHUNK)
            pltpu.sync_copy(pos_hbm.at[pl.ds(off, CHUNK)], pos_v.at[buf])
            for k in range(PAIRS // LANES):
                ta = pair_codes(pos_v[buf, pl.ds(k * 2 * LANES, LANES)])
                tb = pair_codes(pos_v[buf, pl.ds(k * 2 * LANES + LANES, LANES)])
                ga = vperm(ta, evens)
                gb = vperm(tb, evens)
                pidx_v[buf, k // 8, pl.ds((k % 8) * LANES, LANES)] = jnp.where(
                    lo_half, ga, gb
                )

        def fire_gathers(buf):
            for j in range(IDX_ROWS):
                pltpu.async_copy(
                    ptable_hbm.at[pidx_v.at[buf, j]],
                    rows_v.at[buf, pl.ds(j * 128, 128)],
                    sem_g,
                )

        def drain_gathers(buf):
            for j in range(IDX_ROWS):
                pltpu.make_async_copy(
                    ptable_hbm.at[pidx_v.at[buf, j]],
                    rows_v.at[buf, pl.ds(j * 128, 128)],
                    sem_g,
                ).wait()

        def drain_scatter():
            pltpu.make_async_copy(
                rows_v.at[0], out_hbm.at[pl.ds(0, PAIRS)], sem_o
            ).wait()

        load_and_index(0, 0)
        fire_gathers(0)

        def step(t, carry):
            buf = lax.rem(t, 2)
            nbuf = lax.rem(t + 1, 2)

            @pl.when(t > 0)
            def _():
                drain_scatter()

            @pl.when(t < n_chunks - 1)
            def _():
                load_and_index(t + 1, nbuf)

            drain_gathers(buf)

            @pl.when(t < n_chunks - 1)
            def _():
                fire_gathers(nbuf)
            off2 = pl.multiple_of(base2 + t * PAIRS, PAIRS)
            pltpu.async_copy(
                rows_v.at[buf], out_hbm.at[pl.ds(off2, PAIRS)], sem_o
            )
            return carry

        lax.fori_loop(0, n_chunks, step, 0)
        drain_scatter()

    return body(pos_flat, ptable)


def kernel(position, embedding):
    info = plsc.get_sparse_core_info()
    num_workers = info.num_cores * info.num_subcores
    ptable = jnp.concatenate(
        [
            jnp.broadcast_to(embedding[:, None, :], (PERIOD, PERIOD, EMBED)),
            jnp.broadcast_to(embedding[None, :, :], (PERIOD, PERIOD, EMBED)),
        ],
        axis=-1,
    ).reshape(PERIOD * PERIOD, 2 * EMBED)
    pos_flat = position.reshape(-1)
    out = _sc_lookup(pos_flat, ptable, num_workers)
    return out.reshape(position.shape + (EMBED,))
